# baseline (device time: 212312 ns/iter reference)
import jax
import jax.numpy as jnp
from jax import lax
from jax.experimental import pallas as pl
from jax.experimental.pallas import tpu as pltpu

N_DEV = 16


def kernel(x, w_mat, scale_x, scale_w):
    m_total, k_per = x.shape
    _, n = w_mat.shape
    m_per = m_total // N_DEV

    def body(x_ref, w_ref, sx_ref, sw_ref, out_ref,
             w_bf_ref, send_buf, recv_buf, send_sems, recv_sems, credit_sem):
        my = lax.axis_index("i")
        left = lax.rem(my + N_DEV - 1, N_DEV)
        right = lax.rem(my + 1, N_DEV)

        w_bf_ref[...] = w_ref[...].astype(jnp.bfloat16)

        def chunk_partial(k):
            x_chunk = x_ref[pl.ds(k * m_per, m_per), :].astype(jnp.bfloat16)
            return jnp.dot(x_chunk, w_bf_ref[...],
                           preferred_element_type=jnp.float32)

        k0 = lax.rem(my + N_DEV - 1, N_DEV)
        send_buf[0] = chunk_partial(k0).astype(jnp.bfloat16)

        barrier_sem = pltpu.get_barrier_semaphore()
        for nbr in [left, right]:
            pl.semaphore_signal(barrier_sem, inc=1, device_id=(nbr,),
                                device_id_type=pl.DeviceIdType.MESH)
        pl.semaphore_wait(barrier_sem, 2)

        for s in range(N_DEV - 1):
            slot = s % 2
            if s >= 2:
                pl.semaphore_wait(credit_sem, 1)
            rdma = pltpu.make_async_remote_copy(
                src_ref=send_buf.at[slot],
                dst_ref=recv_buf.at[slot],
                send_sem=send_sems.at[slot],
                recv_sem=recv_sems.at[slot],
                device_id=(right,),
                device_id_type=pl.DeviceIdType.MESH,
            )
            rdma.start()
            rdma.wait()

            if s <= N_DEV - 4:
                pl.semaphore_signal(credit_sem, inc=1, device_id=(left,),
                                    device_id_type=pl.DeviceIdType.MESH)
            k = lax.rem(my + 2 * N_DEV - 2 - s, N_DEV)
            acc = recv_buf[slot].astype(jnp.float32) + chunk_partial(k)
            if s < N_DEV - 2:
                send_buf[(s + 1) % 2] = acc.astype(jnp.bfloat16)
            else:
                out_ref[...] = acc * (sx_ref[0] * sw_ref[0])

    return pl.pallas_call(
        body,
        out_shape=jax.ShapeDtypeStruct((m_per, n), jnp.float32),
        in_specs=[
            pl.BlockSpec(memory_space=pltpu.VMEM),
            pl.BlockSpec(memory_space=pltpu.VMEM),
            pl.BlockSpec(memory_space=pltpu.SMEM),
            pl.BlockSpec(memory_space=pltpu.SMEM),
        ],
        out_specs=pl.BlockSpec(memory_space=pltpu.VMEM),
        scratch_shapes=[
            pltpu.VMEM((k_per, n), jnp.bfloat16),
            pltpu.VMEM((2, m_per, n), jnp.bfloat16),
            pltpu.VMEM((2, m_per, n), jnp.bfloat16),
            pltpu.SemaphoreType.DMA((2,)),
            pltpu.SemaphoreType.DMA((2,)),
            pltpu.SemaphoreType.REGULAR,
        ],
        compiler_params=pltpu.CompilerParams(collective_id=0),
    )(x, w_mat, scale_x, scale_w)


# device time: 142752 ns/iter; 1.4873x vs baseline; 1.4873x over previous
import jax
import jax.numpy as jnp
from jax import lax
from jax.experimental import pallas as pl
from jax.experimental.pallas import tpu as pltpu

N_DEV = 16
N_RINGS = 4


def kernel(x, w_mat, scale_x, scale_w):
    m_total, k_per = x.shape
    _, n = w_mat.shape
    m_per = m_total // N_DEV
    rh = m_per // N_RINGS

    def body(x_ref, w_ref, sx_ref, sw_ref, out_ref,
             w_bf_ref, send_buf, recv_buf, send_sems, recv_sems, credit_sems):
        my = lax.axis_index("i")
        left = lax.rem(my + N_DEV - 1, N_DEV)
        right = lax.rem(my + 1, N_DEV)

        w_bf_ref[...] = w_ref[...].astype(jnp.bfloat16)

        downstream = [right, right, left, left]
        upstream = [left, left, right, right]

        def chunk_rows(k, r):
            x_chunk = x_ref[pl.ds(k * m_per + r * rh, rh), :].astype(jnp.bfloat16)
            return jnp.dot(x_chunk, w_bf_ref[...],
                           preferred_element_type=jnp.float32)

        def send_chunk(s, r):
            if r < 2:
                return lax.rem(my + 2 * N_DEV - 1 - s, N_DEV)
            return lax.rem(my + 1 + s, N_DEV)

        def recv_chunk(s, r):
            if r < 2:
                return lax.rem(my + 2 * N_DEV - 2 - s, N_DEV)
            return lax.rem(my + 2 + s, N_DEV)

        for r in range(N_RINGS):
            send_buf[r, 0] = chunk_rows(send_chunk(0, r), r).astype(jnp.bfloat16)

        barrier_sem = pltpu.get_barrier_semaphore()
        for nbr in [left, right]:
            pl.semaphore_signal(barrier_sem, inc=1, device_id=(nbr,),
                                device_id_type=pl.DeviceIdType.MESH)
        pl.semaphore_wait(barrier_sem, 2)

        for s in range(N_DEV - 1):
            slot = s % 2
            rdmas = []
            for r in range(N_RINGS):
                if s >= 2:
                    pl.semaphore_wait(credit_sems.at[r], 1)
                rdma = pltpu.make_async_remote_copy(
                    src_ref=send_buf.at[r, slot],
                    dst_ref=recv_buf.at[r, slot],
                    send_sem=send_sems.at[r, slot],
                    recv_sem=recv_sems.at[r, slot],
                    device_id=(downstream[r],),
                    device_id_type=pl.DeviceIdType.MESH,
                )
                rdma.start()
                rdmas.append(rdma)

            parts = [chunk_rows(recv_chunk(s, r), r) for r in range(N_RINGS)]

            for r in range(N_RINGS):
                rdmas[r].wait()
                if s <= N_DEV - 4:
                    pl.semaphore_signal(credit_sems.at[r], inc=1,
                                        device_id=(upstream[r],),
                                        device_id_type=pl.DeviceIdType.MESH)
                acc = recv_buf[r, slot].astype(jnp.float32) + parts[r]
                if s < N_DEV - 2:
                    send_buf[r, (s + 1) % 2] = acc.astype(jnp.bfloat16)
                else:
                    out_ref[pl.ds(r * rh, rh), :] = acc * (sx_ref[0] * sw_ref[0])

    return pl.pallas_call(
        body,
        out_shape=jax.ShapeDtypeStruct((m_per, n), jnp.float32),
        in_specs=[
            pl.BlockSpec(memory_space=pltpu.VMEM),
            pl.BlockSpec(memory_space=pltpu.VMEM),
            pl.BlockSpec(memory_space=pltpu.SMEM),
            pl.BlockSpec(memory_space=pltpu.SMEM),
        ],
        out_specs=pl.BlockSpec(memory_space=pltpu.VMEM),
        scratch_shapes=[
            pltpu.VMEM((k_per, n), jnp.bfloat16),
            pltpu.VMEM((N_RINGS, 2, rh, n), jnp.bfloat16),
            pltpu.VMEM((N_RINGS, 2, rh, n), jnp.bfloat16),
            pltpu.SemaphoreType.DMA((N_RINGS, 2)),
            pltpu.SemaphoreType.DMA((N_RINGS, 2)),
            pltpu.SemaphoreType.REGULAR((N_RINGS,)),
        ],
        compiler_params=pltpu.CompilerParams(collective_id=0),
    )(x, w_mat, scale_x, scale_w)


# device time: 104821 ns/iter; 2.0255x vs baseline; 1.3619x over previous
import jax
import jax.numpy as jnp
from jax import lax
from jax.experimental import pallas as pl
from jax.experimental.pallas import tpu as pltpu

N_DEV = 16
N_RINGS = 4


def kernel(x, w_mat, scale_x, scale_w):
    m_total, k_per = x.shape
    _, n = w_mat.shape
    m_per = m_total // N_DEV
    rh = m_per // N_RINGS

    def body(x_ref, w_ref, sx_ref, sw_ref, out_ref,
             w_bf_ref, send_buf, recv_buf, send_sems, recv_sems, credit_sems):
        my = lax.axis_index("i")
        left = lax.rem(my + N_DEV - 1, N_DEV)
        right = lax.rem(my + 1, N_DEV)

        w_bf_ref[...] = w_ref[...].astype(jnp.bfloat16)

        downstream = [right, right, left, left]
        upstream = [left, left, right, right]

        def chunk_rows(k, r):
            x_chunk = x_ref[pl.ds(k * m_per + r * rh, rh), :].astype(jnp.bfloat16)
            return jnp.dot(x_chunk, w_bf_ref[...],
                           preferred_element_type=jnp.float32)

        def send_chunk(s, r):
            if r < 2:
                return lax.rem(my + 2 * N_DEV - 1 - s, N_DEV)
            return lax.rem(my + 1 + s, N_DEV)

        def recv_chunk(s, r):
            if r < 2:
                return lax.rem(my + 2 * N_DEV - 2 - s, N_DEV)
            return lax.rem(my + 2 + s, N_DEV)

        def make_rdma(r, s):
            slot = s % 2
            return pltpu.make_async_remote_copy(
                src_ref=send_buf.at[r, slot],
                dst_ref=recv_buf.at[r, slot],
                send_sem=send_sems.at[r, slot],
                recv_sem=recv_sems.at[r, slot],
                device_id=(downstream[r],),
                device_id_type=pl.DeviceIdType.MESH,
            )

        for r in range(N_RINGS):
            send_buf[r, 0] = chunk_rows(send_chunk(0, r), r).astype(jnp.bfloat16)

        barrier_sem = pltpu.get_barrier_semaphore()
        for nbr in [left, right]:
            pl.semaphore_signal(barrier_sem, inc=1, device_id=(nbr,),
                                device_id_type=pl.DeviceIdType.MESH)
        pl.semaphore_wait(barrier_sem, 2)

        rdmas = [make_rdma(r, 0) for r in range(N_RINGS)]
        for r in range(N_RINGS):
            rdmas[r].start()

        for s in range(N_DEV - 1):
            slot = s % 2
            for r in range(N_RINGS):
                part = chunk_rows(recv_chunk(s, r), r)
                rdmas[r].wait()
                if s <= N_DEV - 4:
                    pl.semaphore_signal(credit_sems.at[r], inc=1,
                                        device_id=(upstream[r],),
                                        device_id_type=pl.DeviceIdType.MESH)
                acc = recv_buf[r, slot].astype(jnp.float32) + part
                if s < N_DEV - 2:
                    send_buf[r, (s + 1) % 2] = acc.astype(jnp.bfloat16)
                    if s + 1 >= 2:
                        pl.semaphore_wait(credit_sems.at[r], 1)
                    rdmas[r] = make_rdma(r, s + 1)
                    rdmas[r].start()
                else:
                    out_ref[pl.ds(r * rh, rh), :] = acc * (sx_ref[0] * sw_ref[0])

    return pl.pallas_call(
        body,
        out_shape=jax.ShapeDtypeStruct((m_per, n), jnp.float32),
        in_specs=[
            pl.BlockSpec(memory_space=pltpu.VMEM),
            pl.BlockSpec(memory_space=pltpu.VMEM),
            pl.BlockSpec(memory_space=pltpu.SMEM),
            pl.BlockSpec(memory_space=pltpu.SMEM),
        ],
        out_specs=pl.BlockSpec(memory_space=pltpu.VMEM),
        scratch_shapes=[
            pltpu.VMEM((k_per, n), jnp.bfloat16),
            pltpu.VMEM((N_RINGS, 2, rh, n), jnp.bfloat16),
            pltpu.VMEM((N_RINGS, 2, rh, n), jnp.bfloat16),
            pltpu.SemaphoreType.DMA((N_RINGS, 2)),
            pltpu.SemaphoreType.DMA((N_RINGS, 2)),
            pltpu.SemaphoreType.REGULAR((N_RINGS,)),
        ],
        compiler_params=pltpu.CompilerParams(collective_id=0),
    )(x, w_mat, scale_x, scale_w)


# device time: 80356 ns/iter; 2.6421x vs baseline; 1.3045x over previous
import jax
import jax.numpy as jnp
from jax import lax
from jax.experimental import pallas as pl
from jax.experimental.pallas import tpu as pltpu

N_DEV = 16
CW_HOPS = 8
CCW_HOPS = 7
N_SUB = 2


def kernel(x, w_mat, scale_x, scale_w):
    m_total, k_per = x.shape
    _, n = w_mat.shape
    m_per = m_total // N_DEV
    k_sub = k_per // N_SUB
    f8 = jnp.float8_e4m3fn

    def body(x_ref, w_ref, sx_ref, sw_ref, out_ref,
             w8_own, tile_buf, xtiles, w_cw, w_ccw, acc_ref,
             a2a_send, a2a_recv, cw_send, cw_recv, ccw_send, ccw_recv):
        my = lax.axis_index("i")
        left = lax.rem(my + N_DEV - 1, N_DEV)
        right = lax.rem(my + 1, N_DEV)

        def a2a_rdma(off):
            return pltpu.make_async_remote_copy(
                src_ref=tile_buf.at[off - 1],
                dst_ref=xtiles.at[off - 1],
                send_sem=a2a_send.at[off - 1],
                recv_sem=a2a_recv.at[off - 1],
                device_id=(lax.rem(my + off, N_DEV),),
                device_id_type=pl.DeviceIdType.MESH,
            )

        def cw_rdma(h, u):
            src = w8_own.at[u] if h == 0 else w_cw.at[h - 1, u]
            return pltpu.make_async_remote_copy(
                src_ref=src, dst_ref=w_cw.at[h, u],
                send_sem=cw_send.at[h, u], recv_sem=cw_recv.at[h, u],
                device_id=(right,), device_id_type=pl.DeviceIdType.MESH,
            )

        def ccw_rdma(h, u):
            src = w8_own.at[u] if h == 0 else w_ccw.at[h - 1, u]
            return pltpu.make_async_remote_copy(
                src_ref=src, dst_ref=w_ccw.at[h, u],
                send_sem=ccw_send.at[h, u], recv_sem=ccw_recv.at[h, u],
                device_id=(left,), device_id_type=pl.DeviceIdType.MESH,
            )

        for u in range(N_SUB):
            w8_own[u] = w_ref[pl.ds(u * k_sub, k_sub), :].astype(f8)
        for off in range(1, N_DEV):
            d = lax.rem(my + off, N_DEV)
            tile_buf[off - 1] = x_ref[pl.ds(d * m_per, m_per), :].astype(f8)

        barrier_sem = pltpu.get_barrier_semaphore()
        for off in range(1, N_DEV):
            pl.semaphore_signal(barrier_sem, inc=1,
                                device_id=(lax.rem(my + off, N_DEV),),
                                device_id_type=pl.DeviceIdType.MESH)
        pl.semaphore_wait(barrier_sem, N_DEV - 1)

        for off in range(1, N_DEV):
            a2a_rdma(off).start()
        for u in range(N_SUB):
            cw_rdma(0, u).start()
            ccw_rdma(0, u).start()

        x_own = x_ref[pl.ds(my * m_per, m_per), :].astype(jnp.bfloat16)
        acc_ref[...] = jnp.dot(x_own, w_ref[...].astype(jnp.bfloat16),
                               preferred_element_type=jnp.float32)

        def gemm_term(tile_slot, w_parts):
            xt = xtiles[tile_slot].astype(jnp.bfloat16)
            t = jnp.dot(xt[:, 0:k_sub], w_parts[0].astype(jnp.bfloat16),
                        preferred_element_type=jnp.float32)
            for u in range(1, N_SUB):
                t = t + jnp.dot(xt[:, u * k_sub:(u + 1) * k_sub],
                                w_parts[u].astype(jnp.bfloat16),
                                preferred_element_type=jnp.float32)
            acc_ref[...] += t

        for h in range(CW_HOPS):
            for u in range(N_SUB):
                cw_rdma(h, u).wait_recv()
            if h + 1 < CW_HOPS:
                for u in range(N_SUB):
                    cw_rdma(h + 1, u).start()
            a2a_rdma(h + 1).wait_recv()
            gemm_term(h, [w_cw[h, u] for u in range(N_SUB)])

            if h < CCW_HOPS:
                for u in range(N_SUB):
                    ccw_rdma(h, u).wait_recv()
                if h + 1 < CCW_HOPS:
                    for u in range(N_SUB):
                        ccw_rdma(h + 1, u).start()
                a2a_rdma(N_DEV - 1 - h).wait_recv()
                gemm_term(N_DEV - 2 - h, [w_ccw[h, u] for u in range(N_SUB)])

        out_ref[...] = acc_ref[...] * (sx_ref[0] * sw_ref[0])

        for off in range(1, N_DEV):
            a2a_rdma(off).wait_send()
        for u in range(N_SUB):
            for h in range(CW_HOPS):
                cw_rdma(h, u).wait_send()
            for h in range(CCW_HOPS):
                ccw_rdma(h, u).wait_send()

    return pl.pallas_call(
        body,
        out_shape=jax.ShapeDtypeStruct((m_per, n), jnp.float32),
        in_specs=[
            pl.BlockSpec(memory_space=pltpu.VMEM),
            pl.BlockSpec(memory_space=pltpu.VMEM),
            pl.BlockSpec(memory_space=pltpu.SMEM),
            pl.BlockSpec(memory_space=pltpu.SMEM),
        ],
        out_specs=pl.BlockSpec(memory_space=pltpu.VMEM),
        scratch_shapes=[
            pltpu.VMEM((N_SUB, k_sub, n), f8),
            pltpu.VMEM((N_DEV - 1, m_per, k_per), f8),
            pltpu.VMEM((N_DEV - 1, m_per, k_per), f8),
            pltpu.VMEM((CW_HOPS, N_SUB, k_sub, n), f8),
            pltpu.VMEM((CCW_HOPS, N_SUB, k_sub, n), f8),
            pltpu.VMEM((m_per, n), jnp.float32),
            pltpu.SemaphoreType.DMA((N_DEV - 1,)),
            pltpu.SemaphoreType.DMA((N_DEV - 1,)),
            pltpu.SemaphoreType.DMA((CW_HOPS, N_SUB)),
            pltpu.SemaphoreType.DMA((CW_HOPS, N_SUB)),
            pltpu.SemaphoreType.DMA((CCW_HOPS, N_SUB)),
            pltpu.SemaphoreType.DMA((CCW_HOPS, N_SUB)),
        ],
        compiler_params=pltpu.CompilerParams(collective_id=0),
    )(x, w_mat, scale_x, scale_w)


# device time: 72958 ns/iter; 2.9101x vs baseline; 1.1014x over previous
import jax
import jax.numpy as jnp
from jax import lax
from jax.experimental import pallas as pl
from jax.experimental.pallas import tpu as pltpu

N_DEV = 16
CW_HOPS = 8
CCW_HOPS = 8
N_SUB = 2


def _cw_subs(h):
    return [0] if h == CW_HOPS - 1 else list(range(N_SUB))


def _ccw_subs(h):
    return [1] if h == CCW_HOPS - 1 else list(range(N_SUB))


_CYCLE = [0, 1, 5, 9, 13, 14, 10, 6, 2, 3, 7, 11, 15, 12, 8, 4]
_NEXT = [0] * N_DEV
_PREV = [0] * N_DEV
for _i, _p in enumerate(_CYCLE):
    _NEXT[_p] = _CYCLE[(_i + 1) % N_DEV]
    _PREV[_p] = _CYCLE[(_i - 1) % N_DEV]


def _lut(idx, table):
    import jax.numpy as _jnp
    val = _jnp.int32(table[N_DEV - 1])
    for p in range(N_DEV - 1):
        val = _jnp.where(idx == p, _jnp.int32(table[p]), val)
    return val


def kernel(x, w_mat, scale_x, scale_w):
    m_total, k_per = x.shape
    _, n = w_mat.shape
    m_per = m_total // N_DEV
    k_sub = k_per // N_SUB
    f8 = jnp.float8_e4m3fn

    def body(x_ref, w_ref, sx_ref, sw_ref, out_ref,
             w8_own, tile_buf, xtiles, w_cw, w_ccw, acc_ref,
             a2a_send, a2a_recv, cw_send, cw_recv, ccw_send, ccw_recv):
        my = lax.axis_index("i")
        right = _lut(my, _NEXT)
        left = _lut(my, _PREV)
        dests = []
        cur = my
        for _ in range(N_DEV - 1):
            cur = _lut(cur, _NEXT)
            dests.append(cur)

        def a2a_rdma(off):
            return pltpu.make_async_remote_copy(
                src_ref=tile_buf.at[off - 1],
                dst_ref=xtiles.at[off - 1],
                send_sem=a2a_send.at[off - 1],
                recv_sem=a2a_recv.at[off - 1],
                device_id=(dests[off - 1],),
                device_id_type=pl.DeviceIdType.MESH,
            )

        def cw_rdma(h, u):
            src = w8_own.at[u] if h == 0 else w_cw.at[h - 1, u]
            return pltpu.make_async_remote_copy(
                src_ref=src, dst_ref=w_cw.at[h, u],
                send_sem=cw_send.at[h, u], recv_sem=cw_recv.at[h, u],
                device_id=(right,), device_id_type=pl.DeviceIdType.MESH,
            )

        def ccw_rdma(h, u):
            src = w8_own.at[u] if h == 0 else w_ccw.at[h - 1, u]
            return pltpu.make_async_remote_copy(
                src_ref=src, dst_ref=w_ccw.at[h, u],
                send_sem=ccw_send.at[h, u], recv_sem=ccw_recv.at[h, u],
                device_id=(left,), device_id_type=pl.DeviceIdType.MESH,
            )

        for u in range(N_SUB):
            w8_own[u] = w_ref[pl.ds(u * k_sub, k_sub), :].astype(f8)
        for off in range(1, N_DEV):
            d = dests[off - 1]
            tile_buf[off - 1] = x_ref[pl.ds(d * m_per, m_per), :].astype(f8)

        barrier_sem = pltpu.get_barrier_semaphore()
        for off in range(1, N_DEV):
            pl.semaphore_signal(barrier_sem, inc=1,
                                device_id=(lax.rem(my + off, N_DEV),),
                                device_id_type=pl.DeviceIdType.MESH)
        pl.semaphore_wait(barrier_sem, N_DEV - 1)

        for u in _cw_subs(0):
            cw_rdma(0, u).start()
        for u in _ccw_subs(0):
            ccw_rdma(0, u).start()
        for off in (1, 2, 3, 15, 14, 13):
            a2a_rdma(off).start()

        x_own = x_ref[pl.ds(my * m_per, m_per), :].astype(jnp.bfloat16)
        acc_ref[...] = jnp.dot(x_own, w_ref[...].astype(jnp.bfloat16),
                               preferred_element_type=jnp.float32)

        def gemm_term(tile_slot, w_parts):
            xt = xtiles[tile_slot].astype(jnp.bfloat16)
            t = jnp.dot(xt[:, 0:k_sub], w_parts[0].astype(jnp.bfloat16),
                        preferred_element_type=jnp.float32)
            for u in range(1, N_SUB):
                t = t + jnp.dot(xt[:, u * k_sub:(u + 1) * k_sub],
                                w_parts[u].astype(jnp.bfloat16),
                                preferred_element_type=jnp.float32)
            acc_ref[...] += t

        for h in range(CW_HOPS):
            for u in _cw_subs(h):
                cw_rdma(h, u).wait_recv()
            if h + 1 < CW_HOPS:
                for u in _cw_subs(h + 1):
                    cw_rdma(h + 1, u).start()
            for u in _ccw_subs(h):
                ccw_rdma(h, u).wait_recv()
            if h + 1 < CCW_HOPS:
                for u in _ccw_subs(h + 1):
                    ccw_rdma(h + 1, u).start()
            if h == 0:
                for off in range(4, 13):
                    a2a_rdma(off).start()

            if h < CW_HOPS - 1:
                a2a_rdma(h + 1).wait_recv()
                gemm_term(h, [w_cw[h, u] for u in range(N_SUB)])
                a2a_rdma(N_DEV - 1 - h).wait_recv()
                gemm_term(N_DEV - 2 - h, [w_ccw[h, u] for u in range(N_SUB)])
            else:
                a2a_rdma(N_DEV // 2).wait_recv()
                xt = xtiles[N_DEV // 2 - 1].astype(jnp.bfloat16)
                t = jnp.dot(xt[:, 0:k_sub], w_cw[h, 0].astype(jnp.bfloat16),
                            preferred_element_type=jnp.float32)
                t = t + jnp.dot(xt[:, k_sub:], w_ccw[h, 1].astype(jnp.bfloat16),
                                preferred_element_type=jnp.float32)
                out_ref[...] = (acc_ref[...] + t) * (sx_ref[0] * sw_ref[0])

        for off in range(1, N_DEV):
            a2a_rdma(off).wait_send()
        for h in range(CW_HOPS):
            for u in _cw_subs(h):
                cw_rdma(h, u).wait_send()
        for h in range(CCW_HOPS):
            for u in _ccw_subs(h):
                ccw_rdma(h, u).wait_send()

    return pl.pallas_call(
        body,
        out_shape=jax.ShapeDtypeStruct((m_per, n), jnp.float32),
        in_specs=[
            pl.BlockSpec(memory_space=pltpu.VMEM),
            pl.BlockSpec(memory_space=pltpu.VMEM),
            pl.BlockSpec(memory_space=pltpu.SMEM),
            pl.BlockSpec(memory_space=pltpu.SMEM),
        ],
        out_specs=pl.BlockSpec(memory_space=pltpu.VMEM),
        scratch_shapes=[
            pltpu.VMEM((N_SUB, k_sub, n), f8),
            pltpu.VMEM((N_DEV - 1, m_per, k_per), f8),
            pltpu.VMEM((N_DEV - 1, m_per, k_per), f8),
            pltpu.VMEM((CW_HOPS, N_SUB, k_sub, n), f8),
            pltpu.VMEM((CCW_HOPS, N_SUB, k_sub, n), f8),
            pltpu.VMEM((m_per, n), jnp.float32),
            pltpu.SemaphoreType.DMA((N_DEV - 1,)),
            pltpu.SemaphoreType.DMA((N_DEV - 1,)),
            pltpu.SemaphoreType.DMA((CW_HOPS, N_SUB)),
            pltpu.SemaphoreType.DMA((CW_HOPS, N_SUB)),
            pltpu.SemaphoreType.DMA((CCW_HOPS, N_SUB)),
            pltpu.SemaphoreType.DMA((CCW_HOPS, N_SUB)),
        ],
        compiler_params=pltpu.CompilerParams(collective_id=0),
    )(x, w_mat, scale_x, scale_w)
